# baseline (speedup 1.0000x reference)
import jax
import jax.numpy as jnp
from jax.experimental import pallas as pl
from jax.experimental.pallas import tpu as pltpu

_N = 4096
_NFEAT = 128
_NHID = 64
_NHEADS = 3
_B = 1024
_NB = _N // _B


def _leaky(v):
    return jnp.where(v >= 0, v, 0.2 * v)


def _fused_body(x_ref, adj_ref, Wg1_ref, Wg2_ref, Wb1_ref, Wb2_ref, r_ref,
                W_all_ref, a_all_ref, hk_ref, out_ref,
                h_scr, haug_scr, xaug_scr, vdst_scr):
    i = pl.program_id(0)
    x_full = x_ref[...]

    @pl.when(i == 0)
    def _init():
        ones_h = jnp.ones((_N, _NFEAT - _NHID), dtype=jnp.bfloat16)
        xaug_scr[:, :_NFEAT] = x_full.astype(jnp.bfloat16)
        xaug_scr[:, _NFEAT:] = jnp.ones((_N, _NFEAT), dtype=jnp.bfloat16)
        for k in range(_NHEADS):
            hk = jnp.dot(x_full, W_all_ref[k], preferred_element_type=jnp.float32)
            h_scr[:, k * _NHID:(k + 1) * _NHID] = hk
            haug_scr[k, :, :_NHID] = hk.astype(jnp.bfloat16)
            haug_scr[k, :, _NHID:] = ones_h
            a_dst = a_all_ref[k:k + 1, _NHID:]
            f_dst = jax.lax.dot_general(a_dst, hk, (((1,), (1,)), ((), ())),
                                        preferred_element_type=jnp.float32)
            vdst_scr[2 * k:2 * k + 1, :] = jnp.exp(-f_dst).astype(jnp.bfloat16)
            vdst_scr[2 * k + 1:2 * k + 2, :] = jnp.exp(-0.2 * f_dst).astype(jnp.bfloat16)

    adj_blk = adj_ref[...]
    adj_bf = adj_blk.astype(jnp.bfloat16)
    x_blk = x_ref[pl.ds(i * _B, _B), :]

    nb_aug = jnp.dot(adj_bf, xaug_scr[...], preferred_element_type=jnp.float32)
    s = nb_aug[:, _NFEAT:_NFEAT + 1]
    nb = nb_aug[:, :_NFEAT] / jnp.maximum(s, 1e-12)

    gamma = _leaky(jnp.dot(x_blk, Wg1_ref[...], preferred_element_type=jnp.float32)
                   + jnp.dot(nb, Wg2_ref[...], preferred_element_type=jnp.float32)) + 1.0
    beta = _leaky(jnp.dot(x_blk, Wb1_ref[...], preferred_element_type=jnp.float32)
                  + jnp.dot(nb, Wb2_ref[...], preferred_element_type=jnp.float32))
    r_v = gamma * r_ref[...] + beta
    out_ref[...] = x_blk + r_v - nb

    adj_sq = adj_ref[:, pl.ds(i * _B, _B)]
    rr = jax.lax.broadcasted_iota(jnp.int32, (_B, _B), 0)
    cc = jax.lax.broadcasted_iota(jnp.int32, (_B, _B), 1)
    eye = jnp.where(rr == cc, 1.0, 0.0)
    adj_diag = jnp.sum(adj_sq * eye, axis=1, keepdims=True)

    for k in range(_NHEADS):
        h_blk = h_scr[pl.ds(i * _B, _B), k * _NHID:(k + 1) * _NHID]
        a2d = a_all_ref[k:k + 1, :]
        f_src = jnp.sum(h_blk * a2d[:, :_NHID], axis=1, keepdims=True)
        f_dst_blk = jnp.sum(h_blk * a2d[:, _NHID:], axis=1, keepdims=True)
        ea_src = jnp.exp(-f_src).astype(jnp.bfloat16)
        eb_src = jnp.exp(-0.2 * f_src).astype(jnp.bfloat16)
        ea_dst = vdst_scr[2 * k:2 * k + 1, :]
        eb_dst = vdst_scr[2 * k + 1:2 * k + 2, :]
        e = adj_bf * jnp.minimum(ea_src * ea_dst, eb_src * eb_dst)
        hp_aug = jnp.dot(e, haug_scr[k], preferred_element_type=jnp.float32)
        s_diag = f_src + f_dst_blk
        corr = (1.0 - adj_diag) * jnp.exp(-jnp.maximum(s_diag, 0.2 * s_diag))
        num = hp_aug[:, :_NHID] + corr * h_blk
        den = hp_aug[:, _NHID:_NHID + 1] + corr
        hp = num / den
        hk_ref[:, k * _NHID:(k + 1) * _NHID] = jnp.where(
            hp > 0, hp, jnp.exp(jnp.minimum(hp, 0.0)) - 1.0)


def kernel(x, adj, head, Wg1, Wg2, Wb1, Wb2, r_param, W0, a0, W1, a1, W2, a2):
    del head
    W_all = jnp.stack([W0, W1, W2])
    a_all = jnp.concatenate([a0, a1, a2], axis=0)

    h_k, output = pl.pallas_call(
        _fused_body,
        grid=(_NB,),
        in_specs=[
            pl.BlockSpec((_N, _NFEAT), lambda i: (0, 0)),
            pl.BlockSpec((_B, _N), lambda i: (i, 0)),
            pl.BlockSpec((_NFEAT, _NFEAT), lambda i: (0, 0)),
            pl.BlockSpec((_NFEAT, _NFEAT), lambda i: (0, 0)),
            pl.BlockSpec((_NFEAT, _NFEAT), lambda i: (0, 0)),
            pl.BlockSpec((_NFEAT, _NFEAT), lambda i: (0, 0)),
            pl.BlockSpec((1, _NFEAT), lambda i: (0, 0)),
            pl.BlockSpec((_NHEADS, _NFEAT, _NHID), lambda i: (0, 0, 0)),
            pl.BlockSpec((_NHEADS, 2 * _NHID), lambda i: (0, 0)),
        ],
        out_specs=[
            pl.BlockSpec((_B, _NHEADS * _NHID), lambda i: (i, 0)),
            pl.BlockSpec((_B, _NFEAT), lambda i: (i, 0)),
        ],
        out_shape=[
            jax.ShapeDtypeStruct((_N, _NHEADS * _NHID), jnp.float32),
            jax.ShapeDtypeStruct((_N, _NFEAT), jnp.float32),
        ],
        scratch_shapes=[
            pltpu.VMEM((_N, _NHEADS * _NHID), jnp.float32),
            pltpu.VMEM((_NHEADS, _N, _NFEAT), jnp.bfloat16),
            pltpu.VMEM((_N, 2 * _NFEAT), jnp.bfloat16),
            pltpu.VMEM((8, _N), jnp.bfloat16),
        ],
        compiler_params=pltpu.CompilerParams(
            vmem_limit_bytes=100 * 1024 * 1024,
        ),
    )(x, adj, Wg1, Wg2, Wb1, Wb2, r_param, W_all, a_all)
    return (h_k, output)


# speedup vs baseline: 1.0664x; 1.0664x over previous
import jax
import jax.numpy as jnp
from jax.experimental import pallas as pl
from jax.experimental.pallas import tpu as pltpu

_N = 4096
_NFEAT = 128
_NHID = 64
_NHEADS = 3
_B = 512
_NB = _N // _B


def _leaky(v):
    return jnp.where(v >= 0, v, 0.2 * v)


def _fused_body(x_ref, adj_ref, Wg1_ref, Wg2_ref, Wb1_ref, Wb2_ref, r_ref,
                W_all_ref, a_all_ref, hk_ref, out_ref,
                h_scr, haug_scr, xaug_scr, vdst_scr):
    i = pl.program_id(0)
    x_full = x_ref[...]

    @pl.when(i == 0)
    def _init():
        ones_h = jnp.ones((_N, _NFEAT - _NHID), dtype=jnp.bfloat16)
        xaug_scr[:, :_NFEAT] = x_full.astype(jnp.bfloat16)
        xaug_scr[:, _NFEAT:] = jnp.ones((_N, _NFEAT), dtype=jnp.bfloat16)
        for k in range(_NHEADS):
            hk = jnp.dot(x_full, W_all_ref[k], preferred_element_type=jnp.float32)
            h_scr[:, k * _NHID:(k + 1) * _NHID] = hk
            haug_scr[k, :, :_NHID] = hk.astype(jnp.bfloat16)
            haug_scr[k, :, _NHID:] = ones_h
            a_dst = a_all_ref[k:k + 1, _NHID:]
            f_dst = jax.lax.dot_general(a_dst, hk, (((1,), (1,)), ((), ())),
                                        preferred_element_type=jnp.float32)
            vdst_scr[2 * k:2 * k + 1, :] = jnp.exp(-f_dst).astype(jnp.bfloat16)
            vdst_scr[2 * k + 1:2 * k + 2, :] = jnp.exp(-0.2 * f_dst).astype(jnp.bfloat16)

    adj_blk = adj_ref[...]
    adj_bf = adj_blk.astype(jnp.bfloat16)
    x_blk = x_ref[pl.ds(i * _B, _B), :]

    nb_aug = jnp.dot(adj_bf, xaug_scr[...], preferred_element_type=jnp.float32)
    s = nb_aug[:, _NFEAT:_NFEAT + 1]
    nb = nb_aug[:, :_NFEAT] / jnp.maximum(s, 1e-12)

    gamma = _leaky(jnp.dot(x_blk, Wg1_ref[...], preferred_element_type=jnp.float32)
                   + jnp.dot(nb, Wg2_ref[...], preferred_element_type=jnp.float32)) + 1.0
    beta = _leaky(jnp.dot(x_blk, Wb1_ref[...], preferred_element_type=jnp.float32)
                  + jnp.dot(nb, Wb2_ref[...], preferred_element_type=jnp.float32))
    r_v = gamma * r_ref[...] + beta
    out_ref[...] = x_blk + r_v - nb

    adj_sq = adj_ref[:, pl.ds(i * _B, _B)]
    rr = jax.lax.broadcasted_iota(jnp.int32, (_B, _B), 0)
    cc = jax.lax.broadcasted_iota(jnp.int32, (_B, _B), 1)
    eye = jnp.where(rr == cc, 1.0, 0.0)
    adj_diag = jnp.sum(adj_sq * eye, axis=1, keepdims=True)

    for k in range(_NHEADS):
        h_blk = h_scr[pl.ds(i * _B, _B), k * _NHID:(k + 1) * _NHID]
        a2d = a_all_ref[k:k + 1, :]
        f_src = jnp.sum(h_blk * a2d[:, :_NHID], axis=1, keepdims=True)
        f_dst_blk = jnp.sum(h_blk * a2d[:, _NHID:], axis=1, keepdims=True)
        ea_src = jnp.exp(-f_src).astype(jnp.bfloat16)
        eb_src = jnp.exp(-0.2 * f_src).astype(jnp.bfloat16)
        ea_dst = vdst_scr[2 * k:2 * k + 1, :]
        eb_dst = vdst_scr[2 * k + 1:2 * k + 2, :]
        e = adj_bf * jnp.minimum(ea_src * ea_dst, eb_src * eb_dst)
        hp_aug = jnp.dot(e, haug_scr[k], preferred_element_type=jnp.float32)
        s_diag = f_src + f_dst_blk
        corr = (1.0 - adj_diag) * jnp.exp(-jnp.maximum(s_diag, 0.2 * s_diag))
        num = hp_aug[:, :_NHID] + corr * h_blk
        den = hp_aug[:, _NHID:_NHID + 1] + corr
        hp = num / den
        hk_ref[:, k * _NHID:(k + 1) * _NHID] = jnp.where(
            hp > 0, hp, jnp.exp(jnp.minimum(hp, 0.0)) - 1.0)


def kernel(x, adj, head, Wg1, Wg2, Wb1, Wb2, r_param, W0, a0, W1, a1, W2, a2):
    del head
    W_all = jnp.stack([W0, W1, W2])
    a_all = jnp.concatenate([a0, a1, a2], axis=0)

    h_k, output = pl.pallas_call(
        _fused_body,
        grid=(_NB,),
        in_specs=[
            pl.BlockSpec((_N, _NFEAT), lambda i: (0, 0)),
            pl.BlockSpec((_B, _N), lambda i: (i, 0)),
            pl.BlockSpec((_NFEAT, _NFEAT), lambda i: (0, 0)),
            pl.BlockSpec((_NFEAT, _NFEAT), lambda i: (0, 0)),
            pl.BlockSpec((_NFEAT, _NFEAT), lambda i: (0, 0)),
            pl.BlockSpec((_NFEAT, _NFEAT), lambda i: (0, 0)),
            pl.BlockSpec((1, _NFEAT), lambda i: (0, 0)),
            pl.BlockSpec((_NHEADS, _NFEAT, _NHID), lambda i: (0, 0, 0)),
            pl.BlockSpec((_NHEADS, 2 * _NHID), lambda i: (0, 0)),
        ],
        out_specs=[
            pl.BlockSpec((_B, _NHEADS * _NHID), lambda i: (i, 0)),
            pl.BlockSpec((_B, _NFEAT), lambda i: (i, 0)),
        ],
        out_shape=[
            jax.ShapeDtypeStruct((_N, _NHEADS * _NHID), jnp.float32),
            jax.ShapeDtypeStruct((_N, _NFEAT), jnp.float32),
        ],
        scratch_shapes=[
            pltpu.VMEM((_N, _NHEADS * _NHID), jnp.float32),
            pltpu.VMEM((_NHEADS, _N, _NFEAT), jnp.bfloat16),
            pltpu.VMEM((_N, 2 * _NFEAT), jnp.bfloat16),
            pltpu.VMEM((8, _N), jnp.bfloat16),
        ],
        compiler_params=pltpu.CompilerParams(
            vmem_limit_bytes=100 * 1024 * 1024,
        ),
    )(x, adj, Wg1, Wg2, Wb1, Wb2, r_param, W_all, a_all)
    return (h_k, output)


# speedup vs baseline: 1.0719x; 1.0051x over previous
import jax
import jax.numpy as jnp
from jax.experimental import pallas as pl
from jax.experimental.pallas import tpu as pltpu

_N = 4096
_NFEAT = 128
_NHID = 64
_NHEADS = 3
_B = 512
_NB = _N // _B


def _leaky(v):
    return jnp.where(v >= 0, v, 0.2 * v)


def _fused_body(x_ref, adj_ref, Wg1_ref, Wg2_ref, Wb1_ref, Wb2_ref, r_ref,
                W_all_ref, a_all_ref, hk_ref, out_ref,
                h_scr, haug_scr, xaug_scr, vdst_scr):
    i = pl.program_id(0)
    x_full = x_ref[...]

    @pl.when(i == 0)
    def _init():
        ones_h = jnp.ones((_N, _NFEAT - _NHID), dtype=jnp.bfloat16)
        xaug_scr[:, :_NFEAT] = x_full.astype(jnp.bfloat16)
        xaug_scr[:, _NFEAT:] = jnp.ones((_N, _NFEAT), dtype=jnp.bfloat16)
        for k in range(_NHEADS):
            hk = jnp.dot(x_full, W_all_ref[k], preferred_element_type=jnp.float32)
            h_scr[:, k * _NHID:(k + 1) * _NHID] = hk
            haug_scr[k, :, :_NHID] = hk.astype(jnp.bfloat16)
            haug_scr[k, :, _NHID:] = ones_h
            a_dst = a_all_ref[k:k + 1, _NHID:]
            f_dst = jax.lax.dot_general(a_dst, hk, (((1,), (1,)), ((), ())),
                                        preferred_element_type=jnp.float32)
            vdst_scr[2 * k:2 * k + 1, :] = jnp.exp(-f_dst).astype(jnp.bfloat16)
            vdst_scr[2 * k + 1:2 * k + 2, :] = jnp.exp(-0.2 * f_dst).astype(jnp.bfloat16)

    adj_blk = adj_ref[...]
    adj_bf = adj_blk.astype(jnp.bfloat16)
    x_blk = x_ref[pl.ds(i * _B, _B), :]

    nb_aug = jnp.dot(adj_bf, xaug_scr[...], preferred_element_type=jnp.float32)
    s = nb_aug[:, _NFEAT:_NFEAT + 1]
    nb = nb_aug[:, :_NFEAT] / jnp.maximum(s, 1e-12)

    gamma = _leaky(jnp.dot(x_blk, Wg1_ref[...], preferred_element_type=jnp.float32)
                   + jnp.dot(nb, Wg2_ref[...], preferred_element_type=jnp.float32)) + 1.0
    beta = _leaky(jnp.dot(x_blk, Wb1_ref[...], preferred_element_type=jnp.float32)
                  + jnp.dot(nb, Wb2_ref[...], preferred_element_type=jnp.float32))
    r_v = gamma * r_ref[...] + beta
    out_ref[...] = x_blk + r_v - nb

    adj_sq = adj_ref[:, pl.ds(i * _B, _B)]
    rr = jax.lax.broadcasted_iota(jnp.int32, (_B, _B), 0)
    cc = jax.lax.broadcasted_iota(jnp.int32, (_B, _B), 1)
    eye = jnp.where(rr == cc, 1.0, 0.0)
    adj_diag = jnp.sum(adj_sq * eye, axis=1, keepdims=True)

    for k in range(_NHEADS):
        h_blk = h_scr[pl.ds(i * _B, _B), k * _NHID:(k + 1) * _NHID]
        a2d = a_all_ref[k:k + 1, :]
        f_src = jnp.sum(h_blk * a2d[:, :_NHID], axis=1, keepdims=True)
        f_dst_blk = jnp.sum(h_blk * a2d[:, _NHID:], axis=1, keepdims=True)
        r_row = jnp.exp(0.8 * f_src).astype(jnp.bfloat16)
        ea_dst = vdst_scr[2 * k:2 * k + 1, :]
        eb_dst = vdst_scr[2 * k + 1:2 * k + 2, :]
        e = adj_bf * jnp.minimum(ea_dst, r_row * eb_dst)
        hp_aug = jnp.dot(e, haug_scr[k], preferred_element_type=jnp.float32)
        s_diag = f_src + f_dst_blk
        corr = (1.0 - adj_diag) * jnp.exp(f_src - jnp.maximum(s_diag, 0.2 * s_diag))
        num = hp_aug[:, :_NHID] + corr * h_blk
        den = hp_aug[:, _NHID:_NHID + 1] + corr
        hp = num / den
        hk_ref[:, k * _NHID:(k + 1) * _NHID] = jnp.where(
            hp > 0, hp, jnp.exp(jnp.minimum(hp, 0.0)) - 1.0)


def kernel(x, adj, head, Wg1, Wg2, Wb1, Wb2, r_param, W0, a0, W1, a1, W2, a2):
    del head
    W_all = jnp.stack([W0, W1, W2])
    a_all = jnp.concatenate([a0, a1, a2], axis=0)

    h_k, output = pl.pallas_call(
        _fused_body,
        grid=(_NB,),
        in_specs=[
            pl.BlockSpec((_N, _NFEAT), lambda i: (0, 0)),
            pl.BlockSpec((_B, _N), lambda i: (i, 0)),
            pl.BlockSpec((_NFEAT, _NFEAT), lambda i: (0, 0)),
            pl.BlockSpec((_NFEAT, _NFEAT), lambda i: (0, 0)),
            pl.BlockSpec((_NFEAT, _NFEAT), lambda i: (0, 0)),
            pl.BlockSpec((_NFEAT, _NFEAT), lambda i: (0, 0)),
            pl.BlockSpec((1, _NFEAT), lambda i: (0, 0)),
            pl.BlockSpec((_NHEADS, _NFEAT, _NHID), lambda i: (0, 0, 0)),
            pl.BlockSpec((_NHEADS, 2 * _NHID), lambda i: (0, 0)),
        ],
        out_specs=[
            pl.BlockSpec((_B, _NHEADS * _NHID), lambda i: (i, 0)),
            pl.BlockSpec((_B, _NFEAT), lambda i: (i, 0)),
        ],
        out_shape=[
            jax.ShapeDtypeStruct((_N, _NHEADS * _NHID), jnp.float32),
            jax.ShapeDtypeStruct((_N, _NFEAT), jnp.float32),
        ],
        scratch_shapes=[
            pltpu.VMEM((_N, _NHEADS * _NHID), jnp.float32),
            pltpu.VMEM((_NHEADS, _N, _NFEAT), jnp.bfloat16),
            pltpu.VMEM((_N, 2 * _NFEAT), jnp.bfloat16),
            pltpu.VMEM((8, _N), jnp.bfloat16),
        ],
        compiler_params=pltpu.CompilerParams(
            vmem_limit_bytes=100 * 1024 * 1024,
        ),
    )(x, adj, Wg1, Wg2, Wb1, Wb2, r_param, W_all, a_all)
    return (h_k, output)


# speedup vs baseline: 1.0802x; 1.0078x over previous
import jax
import jax.numpy as jnp
from jax.experimental import pallas as pl
from jax.experimental.pallas import tpu as pltpu

_N = 4096
_NFEAT = 128
_NHID = 64
_NHEADS = 3
_B = 512
_NB = _N // _B


def _leaky(v):
    return jnp.where(v >= 0, v, 0.2 * v)


def _fused_body(x_ref, adj_ref, Wg1_ref, Wg2_ref, Wb1_ref, Wb2_ref, r_ref,
                W_all_ref, a_all_ref, hk_ref, out_ref,
                h_scr, haug_scr, xaug_scr, vdst_scr):
    i = pl.program_id(0)
    x_full = x_ref[...]

    @pl.when(i == 0)
    def _init():
        ones_h = jnp.ones((_N, _NFEAT - _NHID), dtype=jnp.bfloat16)
        xaug_scr[...] = x_full.astype(jnp.bfloat16)
        for k in range(_NHEADS):
            hk = jnp.dot(x_full, W_all_ref[k], preferred_element_type=jnp.float32)
            h_scr[:, k * _NHID:(k + 1) * _NHID] = hk
            haug_scr[k, :, :_NHID] = hk.astype(jnp.bfloat16)
            haug_scr[k, :, _NHID:] = ones_h
            a_dst = a_all_ref[k:k + 1, _NHID:]
            f_dst = jax.lax.dot_general(a_dst, hk, (((1,), (1,)), ((), ())),
                                        preferred_element_type=jnp.float32)
            vdst_scr[2 * k:2 * k + 1, :] = jnp.exp(-f_dst).astype(jnp.bfloat16)
            vdst_scr[2 * k + 1:2 * k + 2, :] = jnp.exp(-0.2 * f_dst).astype(jnp.bfloat16)

    adj_blk = adj_ref[...]
    adj_bf = adj_blk.astype(jnp.bfloat16)
    x_blk = x_ref[pl.ds(i * _B, _B), :]

    s = jnp.sum(adj_blk, axis=1, keepdims=True)
    nb = jnp.dot(adj_bf, xaug_scr[...], preferred_element_type=jnp.float32)
    nb = nb / jnp.maximum(s, 1e-12)

    gamma = _leaky(jnp.dot(x_blk, Wg1_ref[...], preferred_element_type=jnp.float32)
                   + jnp.dot(nb, Wg2_ref[...], preferred_element_type=jnp.float32)) + 1.0
    beta = _leaky(jnp.dot(x_blk, Wb1_ref[...], preferred_element_type=jnp.float32)
                  + jnp.dot(nb, Wb2_ref[...], preferred_element_type=jnp.float32))
    r_v = gamma * r_ref[...] + beta
    out_ref[...] = x_blk + r_v - nb

    adj_sq = adj_ref[:, pl.ds(i * _B, _B)]
    rr = jax.lax.broadcasted_iota(jnp.int32, (_B, _B), 0)
    cc = jax.lax.broadcasted_iota(jnp.int32, (_B, _B), 1)
    eye = jnp.where(rr == cc, 1.0, 0.0)
    adj_diag = jnp.sum(adj_sq * eye, axis=1, keepdims=True)

    for k in range(_NHEADS):
        h_blk = h_scr[pl.ds(i * _B, _B), k * _NHID:(k + 1) * _NHID]
        a2d = a_all_ref[k:k + 1, :]
        f_src = jnp.sum(h_blk * a2d[:, :_NHID], axis=1, keepdims=True)
        f_dst_blk = jnp.sum(h_blk * a2d[:, _NHID:], axis=1, keepdims=True)
        r_row = jnp.exp(0.8 * f_src).astype(jnp.bfloat16)
        ea_dst = vdst_scr[2 * k:2 * k + 1, :]
        eb_dst = vdst_scr[2 * k + 1:2 * k + 2, :]
        e = adj_bf * jnp.minimum(ea_dst, r_row * eb_dst)
        hp_aug = jnp.dot(e, haug_scr[k], preferred_element_type=jnp.float32)
        s_diag = f_src + f_dst_blk
        corr = (1.0 - adj_diag) * jnp.exp(f_src - jnp.maximum(s_diag, 0.2 * s_diag))
        num = hp_aug[:, :_NHID] + corr * h_blk
        den = hp_aug[:, _NHID:_NHID + 1] + corr
        hp = num / den
        hk_ref[:, k * _NHID:(k + 1) * _NHID] = jnp.where(
            hp > 0, hp, jnp.exp(jnp.minimum(hp, 0.0)) - 1.0)


def kernel(x, adj, head, Wg1, Wg2, Wb1, Wb2, r_param, W0, a0, W1, a1, W2, a2):
    del head
    W_all = jnp.stack([W0, W1, W2])
    a_all = jnp.concatenate([a0, a1, a2], axis=0)

    h_k, output = pl.pallas_call(
        _fused_body,
        grid=(_NB,),
        in_specs=[
            pl.BlockSpec((_N, _NFEAT), lambda i: (0, 0)),
            pl.BlockSpec((_B, _N), lambda i: (i, 0)),
            pl.BlockSpec((_NFEAT, _NFEAT), lambda i: (0, 0)),
            pl.BlockSpec((_NFEAT, _NFEAT), lambda i: (0, 0)),
            pl.BlockSpec((_NFEAT, _NFEAT), lambda i: (0, 0)),
            pl.BlockSpec((_NFEAT, _NFEAT), lambda i: (0, 0)),
            pl.BlockSpec((1, _NFEAT), lambda i: (0, 0)),
            pl.BlockSpec((_NHEADS, _NFEAT, _NHID), lambda i: (0, 0, 0)),
            pl.BlockSpec((_NHEADS, 2 * _NHID), lambda i: (0, 0)),
        ],
        out_specs=[
            pl.BlockSpec((_B, _NHEADS * _NHID), lambda i: (i, 0)),
            pl.BlockSpec((_B, _NFEAT), lambda i: (i, 0)),
        ],
        out_shape=[
            jax.ShapeDtypeStruct((_N, _NHEADS * _NHID), jnp.float32),
            jax.ShapeDtypeStruct((_N, _NFEAT), jnp.float32),
        ],
        scratch_shapes=[
            pltpu.VMEM((_N, _NHEADS * _NHID), jnp.float32),
            pltpu.VMEM((_NHEADS, _N, _NFEAT), jnp.bfloat16),
            pltpu.VMEM((_N, _NFEAT), jnp.bfloat16),
            pltpu.VMEM((8, _N), jnp.bfloat16),
        ],
        compiler_params=pltpu.CompilerParams(
            vmem_limit_bytes=100 * 1024 * 1024,
        ),
    )(x, adj, Wg1, Wg2, Wb1, Wb2, r_param, W_all, a_all)
    return (h_k, output)
